# baseline (speedup 1.0000x reference)
import functools

import jax
import jax.numpy as jnp
from jax import lax
from jax.experimental import pallas as pl
from jax.experimental.pallas import tpu as pltpu
from jax.experimental.pallas import tpu_sc as plsc

NPIX = 196608
NFREQS = 128
NPTS = 65536

NW = 32
PTS_PER_W = NPTS // NW
PTS_PER_SUB = 32
ROWS_PER_SUB = PTS_PER_SUB * 4
SUBS = PTS_PER_W // PTS_PER_SUB
NBUF = 4
LANES = 16
SLICES = NFREQS // LANES


def _sc_gather(table, idx3, wgt3):
    mesh = plsc.VectorSubcoreMesh(core_axis_name="c", subcore_axis_name="s")

    @functools.partial(
        pl.kernel,
        out_type=jax.ShapeDtypeStruct((NPTS, NFREQS), jnp.float32),
        mesh=mesh,
        scratch_types=[
            pltpu.VMEM((SUBS, ROWS_PER_SUB), jnp.int32),
            pltpu.VMEM((SUBS, ROWS_PER_SUB), jnp.float32),
            pltpu.VMEM((NBUF, ROWS_PER_SUB, NFREQS), jnp.float32),
            pltpu.VMEM((2, PTS_PER_SUB, NFREQS), jnp.float32),
            pltpu.SemaphoreType.DMA,
            pltpu.SemaphoreType.DMA,
            pltpu.SemaphoreType.DMA,
            pltpu.SemaphoreType.DMA,
            pltpu.SemaphoreType.DMA,
            pltpu.SemaphoreType.DMA,
        ],
    )
    def k(table_hbm, idx_hbm, wgt_hbm, out_hbm, idx_v, wgt_v, buf, outb,
          gsem0, gsem1, gsem2, gsem3, osem0, osem1):
        gsems = (gsem0, gsem1, gsem2, gsem3)
        osems = (osem0, osem1)
        wid = lax.axis_index("s") * 2 + lax.axis_index("c")
        base = wid * PTS_PER_W
        pltpu.sync_copy(idx_hbm.at[wid], idx_v)
        pltpu.sync_copy(wgt_hbm.at[wid], wgt_v)

        for u in range(NBUF):
            pltpu.async_copy(table_hbm.at[idx_v.at[u]], buf.at[u], gsems[u])

        def quad_group(tq, carry):
            for u in range(NBUF):
                g = NBUF * tq + u
                ou = u % 2
                pltpu.make_async_copy(
                    table_hbm.at[idx_v.at[g]], buf.at[u], gsems[u]
                ).wait()

                def _wait_out():
                    pltpu.make_async_copy(
                        outb.at[ou],
                        out_hbm.at[pl.ds(base + (g - 2) * PTS_PER_SUB,
                                         PTS_PER_SUB)],
                        osems[ou],
                    ).wait()

                if u < 2:
                    pl.when(tq >= 1)(_wait_out)
                else:
                    _wait_out()

                @plsc.parallel_loop(0, PTS_PER_SUB // 4, unroll=2)
                def _(q, u=u, ou=ou):
                    wv = wgt_v[g, pl.ds(q * LANES, LANES)]
                    for pp in range(4):
                        p = q * 4 + pp
                        w = [
                            jnp.full((LANES,), wv[4 * pp + j],
                                     dtype=jnp.float32)
                            for j in range(4)
                        ]
                        for s in range(SLICES):
                            sl = pl.ds(s * LANES, LANES)
                            a = (w[0] * buf[u, 4 * p + 0, sl]
                                 + w[1] * buf[u, 4 * p + 1, sl])
                            b2 = (w[2] * buf[u, 4 * p + 2, sl]
                                  + w[3] * buf[u, 4 * p + 3, sl])
                            outb[ou, p, sl] = a + b2

                @pl.when(g + NBUF < SUBS)
                def _():
                    pltpu.async_copy(
                        table_hbm.at[idx_v.at[g + NBUF]], buf.at[u], gsems[u]
                    )

                pltpu.async_copy(
                    outb.at[ou],
                    out_hbm.at[pl.ds(base + g * PTS_PER_SUB, PTS_PER_SUB)],
                    osems[ou],
                )
            return carry

        lax.fori_loop(0, SUBS // NBUF, quad_group, 0, unroll=False)

        for ou in range(2):
            pltpu.make_async_copy(
                outb.at[ou],
                out_hbm.at[pl.ds(base + (SUBS - 2 + ou) * PTS_PER_SUB,
                                 PTS_PER_SUB)],
                osems[ou],
            ).wait()

    return k(table, idx3, wgt3)


def kernel(params, inds, wgts, freqs):
    table = params.reshape(NFREQS, NPIX).T
    idx3 = inds.astype(jnp.int32).reshape(NW, SUBS, ROWS_PER_SUB)
    wgt3 = wgts.astype(jnp.float32).reshape(NW, SUBS, ROWS_PER_SUB)
    out = _sc_gather(table, idx3, wgt3)
    return out.T.reshape(1, 1, 1, NFREQS, NPTS)


# speedup vs baseline: 1.0470x; 1.0470x over previous
import functools

import jax
import jax.numpy as jnp
from jax import lax
from jax.experimental import pallas as pl
from jax.experimental.pallas import tpu as pltpu
from jax.experimental.pallas import tpu_sc as plsc

NPIX = 196608
NFREQS = 128
NPTS = 65536

NW = 32
PTS_PER_W = NPTS // NW
PTS_PER_SUB = 64
GPS = 2
ROWS_PER_G = 128
SUBS = PTS_PER_W // PTS_PER_SUB
NIDX = SUBS * GPS
LANES = 16
SLICES = NFREQS // LANES


def _sc_gather(table, idx3, wgt3):
    mesh = plsc.VectorSubcoreMesh(core_axis_name="c", subcore_axis_name="s")

    @functools.partial(
        pl.kernel,
        out_type=jax.ShapeDtypeStruct((NPTS, NFREQS), jnp.float32),
        mesh=mesh,
        scratch_types=[
            pltpu.VMEM((NIDX, ROWS_PER_G), jnp.int32),
            pltpu.VMEM((NIDX, ROWS_PER_G), jnp.float32),
            pltpu.VMEM((2, PTS_PER_SUB * 4, NFREQS), jnp.float32),
            pltpu.VMEM((2, PTS_PER_SUB, NFREQS), jnp.float32),
            pltpu.SemaphoreType.DMA,
            pltpu.SemaphoreType.DMA,
            pltpu.SemaphoreType.DMA,
            pltpu.SemaphoreType.DMA,
        ],
    )
    def k(table_hbm, idx_hbm, wgt_hbm, out_hbm, idx_v, wgt_v, buf, outb,
          gsem0, gsem1, osem0, osem1):
        gsems = (gsem0, gsem1)
        osems = (osem0, osem1)
        wid = lax.axis_index("s") * 2 + lax.axis_index("c")
        base = wid * PTS_PER_W
        pltpu.sync_copy(idx_hbm.at[wid], idx_v)
        pltpu.sync_copy(wgt_hbm.at[wid], wgt_v)

        def start_gather(g, u):
            for h in range(GPS):
                pltpu.async_copy(
                    table_hbm.at[idx_v.at[GPS * g + h]],
                    buf.at[u, pl.ds(h * ROWS_PER_G, ROWS_PER_G)],
                    gsems[u],
                )

        def wait_gather(g, u):
            for h in range(GPS):
                pltpu.make_async_copy(
                    table_hbm.at[idx_v.at[GPS * g + h]],
                    buf.at[u, pl.ds(h * ROWS_PER_G, ROWS_PER_G)],
                    gsems[u],
                ).wait()

        for u in range(2):
            start_gather(u, u)

        def pair_body(tq, carry):
            for u in range(2):
                g = 2 * tq + u
                wait_gather(g, u)

                @pl.when(tq >= 1)
                def _():
                    pltpu.make_async_copy(
                        outb.at[u],
                        out_hbm.at[pl.ds(base + (g - 2) * PTS_PER_SUB,
                                         PTS_PER_SUB)],
                        osems[u],
                    ).wait()

                @plsc.parallel_loop(0, PTS_PER_SUB // 4, unroll=2)
                def _(q, u=u):
                    wv = wgt_v[GPS * g + q // 8, pl.ds((q % 8) * LANES, LANES)]
                    for pp in range(4):
                        p = q * 4 + pp
                        w = [
                            jnp.full((LANES,), wv[4 * pp + j],
                                     dtype=jnp.float32)
                            for j in range(4)
                        ]
                        for s in range(SLICES):
                            sl = pl.ds(s * LANES, LANES)
                            a = (w[0] * buf[u, 4 * p + 0, sl]
                                 + w[1] * buf[u, 4 * p + 1, sl])
                            b2 = (w[2] * buf[u, 4 * p + 2, sl]
                                  + w[3] * buf[u, 4 * p + 3, sl])
                            outb[u, p, sl] = a + b2

                @pl.when(g + 2 < SUBS)
                def _():
                    start_gather(g + 2, u)

                pltpu.async_copy(
                    outb.at[u],
                    out_hbm.at[pl.ds(base + g * PTS_PER_SUB, PTS_PER_SUB)],
                    osems[u],
                )
            return carry

        lax.fori_loop(0, SUBS // 2, pair_body, 0, unroll=False)

        for u in range(2):
            pltpu.make_async_copy(
                outb.at[u],
                out_hbm.at[pl.ds(base + (SUBS - 2 + u) * PTS_PER_SUB,
                                 PTS_PER_SUB)],
                osems[u],
            ).wait()

    return k(table, idx3, wgt3)


def kernel(params, inds, wgts, freqs):
    table = params.reshape(NFREQS, NPIX).T
    idx3 = inds.astype(jnp.int32).reshape(NW, NIDX, ROWS_PER_G)
    wgt3 = wgts.astype(jnp.float32).reshape(NW, NIDX, ROWS_PER_G)
    out = _sc_gather(table, idx3, wgt3)
    return out.T.reshape(1, 1, 1, NFREQS, NPTS)
